# Initial kernel scaffold; baseline (speedup 1.0000x reference)
import functools

import jax
import jax.numpy as jnp
from jax import lax
from jax.experimental import pallas as pl
from jax.experimental.pallas import tpu as pltpu
from jax.experimental.pallas import tpu_sc as plsc

HID = 64
CH = 128
NBUF = 4
LOOKAHEAD = 2


def _gather_rows(flat_ids, table):
    n = flat_ids.shape[0]
    info = plsc.get_sparse_core_info()
    nc = info.num_cores
    nw = nc * info.num_subcores
    nchunk = n // (nw * CH)
    assert n == nw * nchunk * CH and nchunk % NBUF == 0
    ids3 = flat_ids.reshape(nw, nchunk, CH)

    mesh = plsc.VectorSubcoreMesh(core_axis_name="c", subcore_axis_name="s")

    @functools.partial(
        pl.kernel,
        mesh=mesh,
        out_type=jax.ShapeDtypeStruct((n, HID), jnp.float32),
        scratch_types=(
            [pltpu.VMEM((nchunk, CH), jnp.int32)]
            + [pltpu.VMEM((CH, HID), jnp.float32) for _ in range(NBUF)]
            + [pltpu.SemaphoreType.DMA for _ in range(2 * NBUF)]
        ),
    )
    def gather_kernel(ids_hbm, table_hbm, out_hbm, idx_v, *rest):
        bufs = rest[:NBUF]
        gsem = rest[NBUF:2 * NBUF]
        wsem = rest[2 * NBUF:]
        wid = lax.axis_index("s") * nc + lax.axis_index("c")
        base = wid * (nchunk * CH)

        pltpu.sync_copy(ids_hbm.at[wid], idx_v)

        def g_desc(j, b):
            return pltpu.make_async_copy(
                table_hbm.at[idx_v.at[j]], bufs[b], gsem[b])

        def w_desc(j, b):
            return pltpu.make_async_copy(
                bufs[b], out_hbm.at[pl.ds(base + j * CH, CH)], wsem[b])

        for p in range(LOOKAHEAD):
            g_desc(p, p).start()

        def body(g, carry):
            for b in range(NBUF):
                j = g * NBUF + b
                g_desc(j, b).wait()
                w_desc(j, b).start()
                jn = j + LOOKAHEAD
                bn = (b + LOOKAHEAD) % NBUF

                @pl.when(jn < nchunk)
                def _():
                    @pl.when(jn >= NBUF)
                    def _():
                        w_desc(jn - NBUF, bn).wait()
                    g_desc(jn, bn).start()
            return carry

        lax.fori_loop(0, nchunk // NBUF, body, None)

        for b in range(NBUF):
            w_desc(nchunk - NBUF + b, b).wait()

    return gather_kernel(ids3, table)


def _finish(tok, rids, sids, pos_rows, rt, st, gamma, beta):
    btot, s, _ = tok.shape
    bb = 32
    nr = rt.shape[0]
    ns = st.shape[0]
    na = nr * ns

    def body(tok_ref, r_ref, t_ref, pos_ref, rt_ref, st_ref, g_ref, b_ref,
             o_ref):
        x = tok_ref[...] + pos_ref[...][None, :, :]
        comb = (rt_ref[...][:, None, :] + st_ref[...][None, :, :]).reshape(
            na, HID)
        a = r_ref[...] * ns + t_ref[...]
        oh = (a[..., None] == lax.broadcasted_iota(
            jnp.int32, (bb, s, na), 2)).astype(jnp.float32)
        x = x + lax.dot_general(
            oh, comb, (((2,), (0,)), ((), ())),
            preferred_element_type=jnp.float32)
        mean = jnp.mean(x, axis=-1, keepdims=True)
        xc = x - mean
        var = jnp.mean(xc * xc, axis=-1, keepdims=True)
        y = xc * lax.rsqrt(var + 1e-5)
        o_ref[...] = y * g_ref[...][None, None, :] + b_ref[...][None, None, :]

    return pl.pallas_call(
        body,
        grid=(btot // bb,),
        in_specs=[
            pl.BlockSpec((bb, s, HID), lambda i: (i, 0, 0)),
            pl.BlockSpec((bb, s), lambda i: (i, 0)),
            pl.BlockSpec((bb, s), lambda i: (i, 0)),
            pl.BlockSpec((s, HID), lambda i: (0, 0)),
            pl.BlockSpec((nr, HID), lambda i: (0, 0)),
            pl.BlockSpec((ns, HID), lambda i: (0, 0)),
            pl.BlockSpec((HID,), lambda i: (0,)),
            pl.BlockSpec((HID,), lambda i: (0,)),
        ],
        out_specs=pl.BlockSpec((bb, s, HID), lambda i: (i, 0, 0)),
        out_shape=jax.ShapeDtypeStruct((btot, s, HID), jnp.float32),
    )(tok, rids, sids, pos_rows, rt, st, gamma, beta)


def kernel(input_ids, reasoning_ids, step_positions, token_table, pos_table,
           reasoning_table, step_table, ln_gamma, ln_beta):
    b, s = input_ids.shape
    ids = input_ids.astype(jnp.int32).reshape(-1)
    tok = _gather_rows(ids, token_table).reshape(b, s, HID)
    pos_rows = lax.slice_in_dim(pos_table, 0, s, axis=0)
    return _finish(tok, reasoning_ids.astype(jnp.int32),
                   step_positions.astype(jnp.int32), pos_rows,
                   reasoning_table, step_table, ln_gamma, ln_beta)


# speedup vs baseline: 3.7436x; 3.7436x over previous
import functools

import jax
import jax.numpy as jnp
from jax import lax
from jax.experimental import pallas as pl
from jax.experimental.pallas import tpu as pltpu
from jax.experimental.pallas import tpu_sc as plsc

HID = 64
CH = 128
NBUF = 4
LOOKAHEAD = 2


def _gather_rows(flat_ids, table):
    n = flat_ids.shape[0]
    info = plsc.get_sparse_core_info()
    nc = info.num_cores
    nw = nc * info.num_subcores
    nchunk = n // (nw * CH)
    assert n == nw * nchunk * CH and nchunk % NBUF == 0
    ids3 = flat_ids.reshape(nw, nchunk, CH)

    mesh = plsc.VectorSubcoreMesh(core_axis_name="c", subcore_axis_name="s")

    @functools.partial(
        pl.kernel,
        mesh=mesh,
        compiler_params=pltpu.CompilerParams(use_tc_tiling_on_sc=False),
        out_type=jax.ShapeDtypeStruct((n, HID), jnp.float32),
        scratch_types=(
            [pltpu.VMEM((nchunk, CH), jnp.int32)]
            + [pltpu.VMEM((CH, HID), jnp.float32) for _ in range(NBUF)]
            + [pltpu.SemaphoreType.DMA for _ in range(2 * NBUF)]
        ),
    )
    def gather_kernel(ids_hbm, table_hbm, out_hbm, idx_v, *rest):
        bufs = rest[:NBUF]
        gsem = rest[NBUF:2 * NBUF]
        wsem = rest[2 * NBUF:]
        wid = lax.axis_index("s") * nc + lax.axis_index("c")
        base = wid * (nchunk * CH)

        pltpu.sync_copy(ids_hbm.at[wid], idx_v)

        def g_desc(j, b):
            return pltpu.make_async_copy(
                table_hbm.at[idx_v.at[j]], bufs[b], gsem[b])

        def w_desc(j, b):
            return pltpu.make_async_copy(
                bufs[b], out_hbm.at[pl.ds(base + j * CH, CH)], wsem[b])

        for p in range(LOOKAHEAD):
            g_desc(p, p).start()

        def body(g, carry):
            for b in range(NBUF):
                j = g * NBUF + b
                g_desc(j, b).wait()
                w_desc(j, b).start()
                jn = j + LOOKAHEAD
                bn = (b + LOOKAHEAD) % NBUF

                @pl.when(jn < nchunk)
                def _():
                    @pl.when(jn >= NBUF)
                    def _():
                        w_desc(jn - NBUF, bn).wait()
                    g_desc(jn, bn).start()
            return carry

        lax.fori_loop(0, nchunk // NBUF, body, None)

        for b in range(NBUF):
            w_desc(nchunk - NBUF + b, b).wait()

    return gather_kernel(ids3, table)


def _finish(tok, rids, sids, pos_rows, rt, st, gamma, beta):
    btot, s, _ = tok.shape
    bb = 32
    nr = rt.shape[0]
    ns = st.shape[0]
    na = nr * ns

    def body(tok_ref, r_ref, t_ref, pos_ref, rt_ref, st_ref, g_ref, b_ref,
             o_ref):
        x = tok_ref[...] + pos_ref[...][None, :, :]
        comb = (rt_ref[...][:, None, :] + st_ref[...][None, :, :]).reshape(
            na, HID)
        a = r_ref[...] * ns + t_ref[...]
        oh = (a[..., None] == lax.broadcasted_iota(
            jnp.int32, (bb, s, na), 2)).astype(jnp.float32)
        x = x + lax.dot_general(
            oh, comb, (((2,), (0,)), ((), ())),
            preferred_element_type=jnp.float32)
        mean = jnp.mean(x, axis=-1, keepdims=True)
        xc = x - mean
        var = jnp.mean(xc * xc, axis=-1, keepdims=True)
        y = xc * lax.rsqrt(var + 1e-5)
        o_ref[...] = y * g_ref[...][None, None, :] + b_ref[...][None, None, :]

    return pl.pallas_call(
        body,
        grid=(btot // bb,),
        in_specs=[
            pl.BlockSpec((bb, s, HID), lambda i: (i, 0, 0)),
            pl.BlockSpec((bb, s), lambda i: (i, 0)),
            pl.BlockSpec((bb, s), lambda i: (i, 0)),
            pl.BlockSpec((s, HID), lambda i: (0, 0)),
            pl.BlockSpec((nr, HID), lambda i: (0, 0)),
            pl.BlockSpec((ns, HID), lambda i: (0, 0)),
            pl.BlockSpec((HID,), lambda i: (0,)),
            pl.BlockSpec((HID,), lambda i: (0,)),
        ],
        out_specs=pl.BlockSpec((bb, s, HID), lambda i: (i, 0, 0)),
        out_shape=jax.ShapeDtypeStruct((btot, s, HID), jnp.float32),
    )(tok, rids, sids, pos_rows, rt, st, gamma, beta)


def kernel(input_ids, reasoning_ids, step_positions, token_table, pos_table,
           reasoning_table, step_table, ln_gamma, ln_beta):
    b, s = input_ids.shape
    ids = input_ids.astype(jnp.int32).reshape(-1)
    tok = _gather_rows(ids, token_table).reshape(b, s, HID)
    pos_rows = lax.slice_in_dim(pos_table, 0, s, axis=0)
    return _finish(tok, reasoning_ids.astype(jnp.int32),
                   step_positions.astype(jnp.int32), pos_rows,
                   reasoning_table, step_table, ln_gamma, ln_beta)


# speedup vs baseline: 4.5485x; 1.2150x over previous
import functools

import jax
import jax.numpy as jnp
from jax import lax
from jax.experimental import pallas as pl
from jax.experimental.pallas import tpu as pltpu
from jax.experimental.pallas import tpu_sc as plsc

HID = 64
CH = 128
NBUF = 4
LOOKAHEAD = 2


def _gather_rows(flat_ids, table):
    n = flat_ids.shape[0]
    info = plsc.get_sparse_core_info()
    nc = info.num_cores
    nw = nc * info.num_subcores
    nchunk = n // (nw * CH)
    assert n == nw * nchunk * CH and nchunk % NBUF == 0
    ids3 = flat_ids.reshape(nw, nchunk, CH)

    mesh = plsc.VectorSubcoreMesh(core_axis_name="c", subcore_axis_name="s")

    @functools.partial(
        pl.kernel,
        mesh=mesh,
        compiler_params=pltpu.CompilerParams(use_tc_tiling_on_sc=False),
        out_type=jax.ShapeDtypeStruct((n, 2 * HID), jnp.float32),
        scratch_types=(
            [pltpu.VMEM((nchunk, CH), jnp.int32)]
            + [pltpu.VMEM((CH, HID), jnp.float32) for _ in range(NBUF)]
            + [pltpu.SemaphoreType.DMA for _ in range(2 * NBUF)]
        ),
    )
    def gather_kernel(ids_hbm, table_hbm, out_hbm, idx_v, *rest):
        bufs = rest[:NBUF]
        gsem = rest[NBUF:2 * NBUF]
        wsem = rest[2 * NBUF:]
        wid = lax.axis_index("s") * nc + lax.axis_index("c")
        base = wid * (nchunk * CH)

        pltpu.sync_copy(ids_hbm.at[wid], idx_v)

        def g_desc(j, b):
            return pltpu.make_async_copy(
                table_hbm.at[idx_v.at[j]], bufs[b], gsem[b])

        def w_desc(j, b):
            return pltpu.make_async_copy(
                bufs[b],
                out_hbm.at[pl.ds(base + j * CH, CH), pl.ds(0, HID)],
                wsem[b])

        for p in range(LOOKAHEAD):
            g_desc(p, p).start()

        def body(g, carry):
            for b in range(NBUF):
                j = g * NBUF + b
                g_desc(j, b).wait()
                w_desc(j, b).start()
                jn = j + LOOKAHEAD
                bn = (b + LOOKAHEAD) % NBUF

                @pl.when(jn < nchunk)
                def _():
                    @pl.when(jn >= NBUF)
                    def _():
                        w_desc(jn - NBUF, bn).wait()
                    g_desc(jn, bn).start()
            return carry

        lax.fori_loop(0, nchunk // NBUF, body, None)

        for b in range(NBUF):
            w_desc(nchunk - NBUF + b, b).wait()

    return gather_kernel(ids3, table)


def _finish(tok, rids, sids, pos_rows, rt, st, gamma, beta):
    btot, s, _ = tok.shape
    bb = 32
    nr = rt.shape[0]
    ns = st.shape[0]
    na = nr * ns

    def body(tok_ref, r_ref, t_ref, pos_ref, rt_ref, st_ref, g_ref, b_ref,
             o_ref):
        x = tok_ref[:, :, 0:HID] + pos_ref[...][None, :, :]
        comb = (rt_ref[...][:, None, :] + st_ref[...][None, :, :]).reshape(
            na, HID)
        a = r_ref[...] * ns + t_ref[...]
        oh = (a[..., None] == lax.broadcasted_iota(
            jnp.int32, (bb, s, na), 2)).astype(jnp.float32)
        x = x + lax.dot_general(
            oh, comb, (((2,), (0,)), ((), ())),
            preferred_element_type=jnp.float32)
        mean = jnp.mean(x, axis=-1, keepdims=True)
        xc = x - mean
        var = jnp.mean(xc * xc, axis=-1, keepdims=True)
        y = xc * lax.rsqrt(var + 1e-5)
        o_ref[...] = y * g_ref[...][None, None, :] + b_ref[...][None, None, :]

    return pl.pallas_call(
        body,
        grid=(btot // bb,),
        in_specs=[
            pl.BlockSpec((bb, s, 2 * HID), lambda i: (i, 0, 0)),
            pl.BlockSpec((bb, s), lambda i: (i, 0)),
            pl.BlockSpec((bb, s), lambda i: (i, 0)),
            pl.BlockSpec((s, HID), lambda i: (0, 0)),
            pl.BlockSpec((nr, HID), lambda i: (0, 0)),
            pl.BlockSpec((ns, HID), lambda i: (0, 0)),
            pl.BlockSpec((HID,), lambda i: (0,)),
            pl.BlockSpec((HID,), lambda i: (0,)),
        ],
        out_specs=pl.BlockSpec((bb, s, HID), lambda i: (i, 0, 0)),
        out_shape=jax.ShapeDtypeStruct((btot, s, HID), jnp.float32),
    )(tok, rids, sids, pos_rows, rt, st, gamma, beta)


def kernel(input_ids, reasoning_ids, step_positions, token_table, pos_table,
           reasoning_table, step_table, ln_gamma, ln_beta):
    b, s = input_ids.shape
    ids = input_ids.astype(jnp.int32).reshape(-1)
    tok = _gather_rows(ids, token_table).reshape(b, s, 2 * HID)
    pos_rows = lax.slice_in_dim(pos_table, 0, s, axis=0)
    return _finish(tok, reasoning_ids.astype(jnp.int32),
                   step_positions.astype(jnp.int32), pos_rows,
                   reasoning_table, step_table, ln_gamma, ln_beta)
